# Initial kernel scaffold; baseline (speedup 1.0000x reference)
import jax
import jax.numpy as jnp
from jax.experimental import pallas as pl


def kernel(inputs, adj, W, W_agg):
    raise NotImplementedError("write your pallas kernel here")


# speedup vs baseline: 1.9915x; 1.9915x over previous
import jax
import jax.numpy as jnp
from jax.experimental import pallas as pl
from jax.experimental.pallas import tpu as pltpu

_N = 10000
_D = 128
_BM = 400


def _presup_kernel(x_ref, w_ref, ps_ref):
    ps_ref[...] = jnp.dot(x_ref[0], w_ref[...],
                          preferred_element_type=jnp.float32)


def _agg_kernel(adj_ref, ps_ref, out_ref):
    acc = jnp.dot(adj_ref[0], ps_ref[...],
                  preferred_element_type=jnp.float32)
    acc = jnp.maximum(acc, 0.0)
    out_ref[0] = acc[:, :_D]
    out_ref[1] = acc[:, _D:]


def kernel(inputs, adj, W, W_agg):
    B, N, D = inputs.shape
    del W_agg

    ps = pl.pallas_call(
        _presup_kernel,
        grid=(B,),
        in_specs=[
            pl.BlockSpec((1, N, D), lambda b: (b, 0, 0)),
            pl.BlockSpec((D, D), lambda b: (0, 0)),
        ],
        out_specs=pl.BlockSpec((N, D), lambda b: (0, b)),
        out_shape=jax.ShapeDtypeStruct((N, B * D), jnp.float32),
        compiler_params=pltpu.CompilerParams(
            dimension_semantics=("arbitrary",)),
    )(inputs, W)

    adj3 = adj.reshape(N // _BM, _BM, N)
    out = pl.pallas_call(
        _agg_kernel,
        grid=(N // _BM,),
        in_specs=[
            pl.BlockSpec((1, _BM, N), lambda i: (i, 0, 0)),
            pl.BlockSpec((N, B * D), lambda i: (0, 0)),
        ],
        out_specs=pl.BlockSpec((B, _BM, D), lambda i: (0, i, 0)),
        out_shape=jax.ShapeDtypeStruct((B, N, D), jnp.float32),
        compiler_params=pltpu.CompilerParams(
            dimension_semantics=("arbitrary",)),
    )(adj3, ps)

    return out


# speedup vs baseline: 2.0930x; 1.0510x over previous
import jax
import jax.numpy as jnp
from jax.experimental import pallas as pl
from jax.experimental.pallas import tpu as pltpu

_BM = 400


def _fused_kernel(x_ref, adj_ref, w_ref, out_ref, ps_ref):
    b, _, d = x_ref.shape

    @pl.when(pl.program_id(0) == 0)
    def _compute_presup():
        for i in range(b):
            ps_ref[:, i * d:(i + 1) * d] = jnp.dot(
                x_ref[i], w_ref[...], preferred_element_type=jnp.float32)

    acc = jnp.dot(adj_ref[0], ps_ref[...],
                  preferred_element_type=jnp.float32)
    acc = jnp.maximum(acc, 0.0)
    for i in range(b):
        out_ref[i] = acc[:, i * d:(i + 1) * d]


def kernel(inputs, adj, W, W_agg):
    B, N, D = inputs.shape
    del W_agg

    adj3 = adj.reshape(N // _BM, _BM, N)
    out = pl.pallas_call(
        _fused_kernel,
        grid=(N // _BM,),
        in_specs=[
            pl.BlockSpec((B, N, D), lambda i: (0, 0, 0)),
            pl.BlockSpec((1, _BM, N), lambda i: (i, 0, 0)),
            pl.BlockSpec((D, D), lambda i: (0, 0)),
        ],
        out_specs=pl.BlockSpec((B, _BM, D), lambda i: (0, i, 0)),
        out_shape=jax.ShapeDtypeStruct((B, N, D), jnp.float32),
        scratch_shapes=[pltpu.VMEM((N, B * D), jnp.float32)],
        compiler_params=pltpu.CompilerParams(
            dimension_semantics=("arbitrary",)),
    )(inputs, adj3, W)

    return out


# speedup vs baseline: 2.0971x; 1.0020x over previous
import jax
import jax.numpy as jnp
from jax.experimental import pallas as pl
from jax.experimental.pallas import tpu as pltpu

_BM = 200


def _fused_kernel(x_ref, adj_ref, w_ref, out_ref, ps_ref):
    b, _, d = x_ref.shape

    @pl.when(pl.program_id(0) == 0)
    def _compute_presup():
        for i in range(b):
            ps_ref[:, i * d:(i + 1) * d] = jnp.dot(
                x_ref[i], w_ref[...], preferred_element_type=jnp.float32)

    acc = jnp.dot(adj_ref[0], ps_ref[...],
                  preferred_element_type=jnp.float32)
    acc = jnp.maximum(acc, 0.0)
    for i in range(b):
        out_ref[i] = acc[:, i * d:(i + 1) * d]


def kernel(inputs, adj, W, W_agg):
    B, N, D = inputs.shape
    del W_agg

    adj3 = adj.reshape(N // _BM, _BM, N)
    out = pl.pallas_call(
        _fused_kernel,
        grid=(N // _BM,),
        in_specs=[
            pl.BlockSpec((B, N, D), lambda i: (0, 0, 0)),
            pl.BlockSpec((1, _BM, N), lambda i: (i, 0, 0)),
            pl.BlockSpec((D, D), lambda i: (0, 0)),
        ],
        out_specs=pl.BlockSpec((B, _BM, D), lambda i: (0, i, 0)),
        out_shape=jax.ShapeDtypeStruct((B, N, D), jnp.float32),
        scratch_shapes=[pltpu.VMEM((N, B * D), jnp.float32)],
        compiler_params=pltpu.CompilerParams(
            dimension_semantics=("arbitrary",)),
    )(inputs, adj3, W)

    return out
